# baseline (speedup 1.0000x reference)
import functools

import jax
import jax.numpy as jnp
from jax import lax
from jax.experimental import pallas as pl
from jax.experimental.pallas import tpu as pltpu
from jax.experimental.pallas import tpu_sc as plsc

NB = 8


def kernel(x, table):
    B, L = x.shape
    V, D = table.shape
    info = plsc.get_sparse_core_info()
    nw = info.num_cores * info.num_subcores
    b_per_w = B // nw
    nb = min(NB, b_per_w)
    n_chunks = b_per_w // nb
    assert n_chunks % 2 == 0 and n_chunks >= 4
    mesh = plsc.VectorSubcoreMesh(core_axis_name="c", subcore_axis_name="s")

    @functools.partial(
        pl.kernel,
        mesh=mesh,
        out_type=jax.ShapeDtypeStruct((B, L, D), jnp.float32),
        scratch_types=[
            pltpu.VMEM((b_per_w, L), jnp.int32),
            pltpu.VMEM((nb, L, D), jnp.float32),
            pltpu.VMEM((nb, L, D), jnp.float32),
            pltpu.SemaphoreType.DMA,
            pltpu.SemaphoreType.DMA,
            pltpu.SemaphoreType.DMA,
            pltpu.SemaphoreType.DMA,
        ],
        compiler_params=pltpu.CompilerParams(use_tc_tiling_on_sc=False),
    )
    def k(idx_hbm, table_hbm, out_hbm, idx2d, rows0, rows1, g0, g1, s0, s1):
        wid = lax.axis_index("s") * info.num_cores + lax.axis_index("c")
        base_b = wid * b_per_w
        rows = (rows0, rows1)
        gsem = (g0, g1)
        ssem = (s0, s1)

        pltpu.sync_copy(idx_hbm.at[pl.ds(base_b, b_per_w)], idx2d)

        def gather(i, b):
            for jj in range(nb):
                pltpu.make_async_copy(
                    table_hbm.at[idx2d.at[i * nb + jj]],
                    rows[b].at[jj], gsem[b],
                ).start()

        def wait_gather(b):
            for jj in range(nb):
                pltpu.make_async_copy(
                    table_hbm.at[idx2d.at[0]], rows[b].at[jj], gsem[b]
                ).wait()

        def store(i, b):
            pltpu.make_async_copy(
                rows[b], out_hbm.at[pl.ds(base_b + i * nb, nb)], ssem[b]
            ).start()

        def wait_store(b):
            pltpu.make_async_copy(
                rows[b], out_hbm.at[pl.ds(0, nb)], ssem[b]
            ).wait()

        gather(0, 0)
        gather(1, 1)

        npairs = n_chunks // 2

        def body(p, carry):
            for b in range(2):
                i = 2 * p + b
                wait_gather(b)
                store(i, b)

                @pl.when(p < npairs - 1)
                def _():
                    wait_store(b)
                    gather(i + 2, b)

            return carry

        lax.fori_loop(0, npairs, body, 0)
        wait_store(0)
        wait_store(1)

    return k(x, table)


# speedup vs baseline: 1.0048x; 1.0048x over previous
import functools

import jax
import jax.numpy as jnp
from jax import lax
from jax.experimental import pallas as pl
from jax.experimental.pallas import tpu as pltpu
from jax.experimental.pallas import tpu_sc as plsc

G = 128
NG = 4


def kernel(x, table):
    B, L = x.shape
    V, D = table.shape
    R = B * L
    info = plsc.get_sparse_core_info()
    nw = info.num_cores * info.num_subcores
    r_per_w = R // nw
    assert r_per_w % G == 0
    nvec = r_per_w // G
    C = NG * G
    n_chunks = r_per_w // C
    assert n_chunks % 2 == 0 and n_chunks >= 4
    mesh = plsc.VectorSubcoreMesh(core_axis_name="c", subcore_axis_name="s")

    @functools.partial(
        pl.kernel,
        mesh=mesh,
        out_type=jax.ShapeDtypeStruct((R, D), jnp.float32),
        scratch_types=[
            pltpu.VMEM((nvec, G), jnp.int32),
            pltpu.VMEM((C, D), jnp.float32),
            pltpu.VMEM((C, D), jnp.float32),
            pltpu.SemaphoreType.DMA,
            pltpu.SemaphoreType.DMA,
            pltpu.SemaphoreType.DMA,
            pltpu.SemaphoreType.DMA,
        ],
        compiler_params=pltpu.CompilerParams(use_tc_tiling_on_sc=False),
    )
    def k(idx_hbm, table_hbm, out_hbm, idx2d, rows0, rows1, g0, g1, s0, s1):
        wid = lax.axis_index("s") * info.num_cores + lax.axis_index("c")
        base_r = wid * r_per_w
        rows = (rows0, rows1)
        gsem = (g0, g1)
        ssem = (s0, s1)

        pltpu.sync_copy(idx_hbm.at[pl.ds(wid * nvec, nvec)], idx2d)

        def gather(i, b):
            for jj in range(NG):
                pltpu.make_async_copy(
                    table_hbm.at[idx2d.at[i * NG + jj]],
                    rows[b].at[pl.ds(jj * G, G)], gsem[b],
                ).start()

        def wait_gather(b):
            for jj in range(NG):
                pltpu.make_async_copy(
                    table_hbm.at[idx2d.at[0]],
                    rows[b].at[pl.ds(0, G)], gsem[b],
                ).wait()

        def store(i, b):
            pltpu.make_async_copy(
                rows[b], out_hbm.at[pl.ds(base_r + i * C, C)], ssem[b]
            ).start()

        def wait_store(b):
            pltpu.make_async_copy(
                rows[b], out_hbm.at[pl.ds(0, C)], ssem[b]
            ).wait()

        gather(0, 0)
        gather(1, 1)

        npairs = n_chunks // 2

        def body(p, carry):
            for b in range(2):
                i = 2 * p + b
                wait_gather(b)
                store(i, b)

                @pl.when(p < npairs - 1)
                def _():
                    wait_store(b)
                    gather(i + 2, b)

            return carry

        lax.fori_loop(0, npairs, body, 0)
        wait_store(0)
        wait_store(1)

    out = k(x.reshape(R // G, G), table)
    return out.reshape(B, L, D)


# speedup vs baseline: 1.0952x; 1.0900x over previous
import functools

import jax
import jax.numpy as jnp
from jax import lax
from jax.experimental import pallas as pl
from jax.experimental.pallas import tpu as pltpu
from jax.experimental.pallas import tpu_sc as plsc

NB = 2
VL = 16


def kernel(x, table):
    B, L = x.shape
    V, D = table.shape
    info = plsc.get_sparse_core_info()
    nw = info.num_cores * info.num_subcores
    b_per_w = B // nw
    nb = min(NB, b_per_w)
    n_chunks = b_per_w // nb
    assert n_chunks % 2 == 0 and n_chunks >= 4
    mesh = plsc.VectorSubcoreMesh(core_axis_name="c", subcore_axis_name="s")

    @functools.partial(
        pl.kernel,
        mesh=mesh,
        out_type=jax.ShapeDtypeStruct((B, L, D), jnp.float32),
        scratch_types=[
            pltpu.VMEM((b_per_w, L), jnp.int32),
            pltpu.VMEM((nb * L, 2 * D), jnp.float32),
            pltpu.VMEM((nb * L, 2 * D), jnp.float32),
            pltpu.VMEM((nb, L, D), jnp.float32),
            pltpu.VMEM((nb, L, D), jnp.float32),
            pltpu.SemaphoreType.DMA,
            pltpu.SemaphoreType.DMA,
            pltpu.SemaphoreType.DMA,
            pltpu.SemaphoreType.DMA,
        ],
        compiler_params=pltpu.CompilerParams(use_tc_tiling_on_sc=True),
    )
    def k(idx_hbm, table_hbm, out_hbm, idx2d,
          rows0, rows1, c0, c1, g0, g1, s0, s1):
        wid = lax.axis_index("s") * info.num_cores + lax.axis_index("c")
        base_b = wid * b_per_w
        rows = (rows0, rows1)
        cbuf = (c0, c1)
        gsem = (g0, g1)
        ssem = (s0, s1)

        pltpu.sync_copy(idx_hbm.at[pl.ds(base_b, b_per_w)], idx2d)

        def gather(i, b):
            for jj in range(nb):
                pltpu.make_async_copy(
                    table_hbm.at[idx2d.at[i * nb + jj]],
                    rows[b].at[pl.ds(jj * L, L)], gsem[b],
                ).start()

        def wait_gather(b):
            for jj in range(nb):
                pltpu.make_async_copy(
                    table_hbm.at[idx2d.at[0]],
                    rows[b].at[pl.ds(0, L)], gsem[b],
                ).wait()

        def compact(b):
            for jj in range(nb):
                @pl.loop(0, L)
                def _(r):
                    for c in range(D // VL):
                        cbuf[b][jj, r, pl.ds(c * VL, VL)] = (
                            rows[b][jj * L + r, pl.ds(c * VL, VL)]
                        )

        def store(i, b):
            pltpu.make_async_copy(
                cbuf[b], out_hbm.at[pl.ds(base_b + i * nb, nb)], ssem[b]
            ).start()

        def wait_store(b):
            pltpu.make_async_copy(
                cbuf[b], out_hbm.at[pl.ds(0, nb)], ssem[b]
            ).wait()

        gather(0, 0)
        gather(1, 1)

        npairs = n_chunks // 2

        def body(p, carry):
            for b in range(2):
                i = 2 * p + b
                wait_gather(b)

                @pl.when(p > 0)
                def _():
                    wait_store(b)

                compact(b)
                store(i, b)

                @pl.when(p < npairs - 1)
                def _():
                    gather(i + 2, b)

            return carry

        lax.fori_loop(0, npairs, body, 0)
        wait_store(0)
        wait_store(1)

    table_pad = jnp.pad(table, ((0, 0), (0, D)))
    return k(x, table_pad)


# speedup vs baseline: 1.1005x; 1.0048x over previous
import functools

import jax
import jax.numpy as jnp
from jax import lax
from jax.experimental import pallas as pl
from jax.experimental.pallas import tpu as pltpu
from jax.experimental.pallas import tpu_sc as plsc

NB = 4
VL = 16


def kernel(x, table):
    B, L = x.shape
    V, D = table.shape
    info = plsc.get_sparse_core_info()
    nw = info.num_cores * info.num_subcores
    b_per_w = B // nw
    nb = NB
    n_chunks = b_per_w // nb
    assert n_chunks % 2 == 0 and n_chunks >= 4
    mesh = plsc.VectorSubcoreMesh(core_axis_name="c", subcore_axis_name="s")

    @functools.partial(
        pl.kernel,
        mesh=mesh,
        out_type=jax.ShapeDtypeStruct((B, L, D), jnp.float32),
        scratch_types=[
            pltpu.VMEM((b_per_w // 2, 2 * L), jnp.int32),
            pltpu.VMEM((nb * L, 2 * D), jnp.float32),
            pltpu.VMEM((nb * L, 2 * D), jnp.float32),
            pltpu.VMEM((2, L, D), jnp.float32),
            pltpu.VMEM((2, L, D), jnp.float32),
            pltpu.SemaphoreType.DMA,
            pltpu.SemaphoreType.DMA,
            pltpu.SemaphoreType.DMA,
            pltpu.SemaphoreType.DMA,
        ],
        compiler_params=pltpu.CompilerParams(use_tc_tiling_on_sc=True),
    )
    def k(idx_hbm, table_hbm, out_hbm, idx2d,
          rows0, rows1, c0, c1, g0, g1, s0, s1):
        wid = lax.axis_index("s") * info.num_cores + lax.axis_index("c")
        base_b = wid * b_per_w
        rows = (rows0, rows1)
        cbuf = (c0, c1)
        gsem = (g0, g1)
        ssem = (s0, s1)

        pltpu.sync_copy(
            idx_hbm.at[pl.ds(wid * (b_per_w // 2), b_per_w // 2)], idx2d
        )

        def gather(i, b):
            for h in range(nb // 2):
                pltpu.make_async_copy(
                    table_hbm.at[idx2d.at[i * (nb // 2) + h]],
                    rows[b].at[pl.ds(h * 2 * L, 2 * L)], gsem[b],
                ).start()

        def wait_gather(b):
            for h in range(nb // 2):
                pltpu.make_async_copy(
                    table_hbm.at[idx2d.at[0]],
                    rows[b].at[pl.ds(0, 2 * L)], gsem[b],
                ).wait()

        def compact(b, h):
            for jj in range(2):
                @pl.loop(0, L)
                def _(r):
                    for c in range(D // VL):
                        cbuf[h][jj, r, pl.ds(c * VL, VL)] = (
                            rows[b][(2 * h + jj) * L + r, pl.ds(c * VL, VL)]
                        )

        def store(i, h):
            pltpu.make_async_copy(
                cbuf[h],
                out_hbm.at[pl.ds(base_b + i * nb + 2 * h, 2)], ssem[h],
            ).start()

        def wait_store(h):
            pltpu.make_async_copy(
                cbuf[h], out_hbm.at[pl.ds(0, 2)], ssem[h]
            ).wait()

        gather(0, 0)
        gather(1, 1)

        npairs = n_chunks // 2

        def body(p, carry):
            for b in range(2):
                i = 2 * p + b
                wait_gather(b)
                for h in range(2):
                    @pl.when(i > 0)
                    def _():
                        wait_store(h)

                    compact(b, h)
                    store(i, h)

                @pl.when(p < npairs - 1)
                def _():
                    gather(i + 2, b)

            return carry

        lax.fori_loop(0, npairs, body, 0)
        wait_store(0)
        wait_store(1)

    table_pad = jnp.pad(table, ((0, 0), (0, D)))
    return k(x.reshape(B // 2, 2 * L), table_pad)
